# baseline (speedup 1.0000x reference)
import functools

import jax
import jax.numpy as jnp
from jax.experimental import pallas as pl
from jax.experimental.pallas import tpu as pltpu


def _mm_kernel(a_ref, b_ref, o_ref):
    acc = jax.lax.dot_general(
        a_ref[...].astype(jnp.bfloat16), b_ref[...].astype(jnp.bfloat16),
        dimension_numbers=(((1,), (0,)), ((), ())),
        preferred_element_type=jnp.float32,
        precision=jax.lax.Precision.DEFAULT,
    )

    @pl.when(pl.program_id(1) == 0)
    def _init():
        o_ref[...] = acc

    @pl.when(pl.program_id(1) != 0)
    def _accum():
        o_ref[...] += acc


@functools.partial(jax.jit, static_argnames=())
def kernel(adj, embeds):
    m, k = adj.shape
    k2, d = embeds.shape
    bm = 512
    bk = 2048
    return pl.pallas_call(
        _mm_kernel,
        grid=(m // bm, k // bk),
        in_specs=[
            pl.BlockSpec((bm, bk), lambda i, j: (i, j)),
            pl.BlockSpec((bk, d), lambda i, j: (j, 0)),
        ],
        out_specs=pl.BlockSpec((bm, d), lambda i, j: (i, 0)),
        out_shape=jax.ShapeDtypeStruct((m, d), jnp.float32),
        compiler_params=pltpu.CompilerParams(
            dimension_semantics=("parallel", "arbitrary"),
        ),
    )(adj, embeds)


# speedup vs baseline: 1.2973x; 1.2973x over previous
import functools

import jax
import jax.numpy as jnp
from jax.experimental import pallas as pl
from jax.experimental.pallas import tpu as pltpu


def _dot(a, b):
    return jax.lax.dot_general(
        a.astype(jnp.bfloat16), b,
        dimension_numbers=(((1,), (0,)), ((), ())),
        preferred_element_type=jnp.float32,
        precision=jax.lax.Precision.DEFAULT,
    )


def _mm_kernel(a0_ref, a1_ref, b_ref, o_ref):
    b16 = b_ref[...].astype(jnp.bfloat16)
    bm = a0_ref.shape[0]
    o_ref[:bm, :] = _dot(a0_ref[...], b16)
    o_ref[bm:, :] = _dot(a1_ref[...], b16)


@functools.partial(jax.jit, static_argnames=())
def kernel(adj, embeds):
    m, k = adj.shape
    k2, d = embeds.shape
    bm = 512
    return pl.pallas_call(
        _mm_kernel,
        grid=(m // (2 * bm),),
        in_specs=[
            pl.BlockSpec((bm, k), lambda i: (2 * i, 0)),
            pl.BlockSpec((bm, k), lambda i: (2 * i + 1, 0)),
            pl.BlockSpec((k, d), lambda i: (0, 0)),
        ],
        out_specs=pl.BlockSpec((2 * bm, d), lambda i: (i, 0)),
        out_shape=jax.ShapeDtypeStruct((m, d), jnp.float32),
        compiler_params=pltpu.CompilerParams(
            dimension_semantics=("arbitrary",),
        ),
    )(adj, adj, embeds)
